# baseline (speedup 1.0000x reference)
import functools

import jax
import jax.numpy as jnp
from jax import lax
from jax.experimental import pallas as pl
from jax.experimental.pallas import tpu as pltpu
from jax.experimental.pallas import tpu_sc as plsc

B = 4096
L = 200
DIM = 32

NC = 2
NS = 16
NW = NC * NS

LT = L // 8
ET = DIM // 8


@functools.partial(
    pl.kernel,
    mesh=plsc.VectorSubcoreMesh(core_axis_name="c", subcore_axis_name="s"),
    out_type=(
        jax.ShapeDtypeStruct((L, ET, NW, 1024), jnp.float32),
        jax.ShapeDtypeStruct((L, ET, NW, 1024), jnp.float32),
    ),
    scratch_types=[
        pltpu.VMEM((LT, 8, 128), jnp.int32),
        pltpu.VMEM((128, DIM), jnp.float32),
        pltpu.VMEM((128, DIM), jnp.float32),
        pltpu.VMEM((ET * 1024,), jnp.float32),
        pltpu.VMEM((ET * 1024,), jnp.float32),
        pltpu.SemaphoreType.DMA,
        pltpu.SemaphoreType.DMA,
        pltpu.SemaphoreType.DMA,
    ],
    compiler_params=pltpu.CompilerParams(use_tc_tiling_on_sc=False,
                                         needs_layout_passes=False),
)
def _gather_kernel(ids0_hbm, ids1_hbm, table_hbm, out0_hbm, out1_hbm,
                   idx_v, g0, g1, t0, t1, gsem, osem, ssem):
    wid = lax.axis_index("s") * NC + lax.axis_index("c")
    iota16 = lax.iota(jnp.int32, 16)
    sc_lo = iota16 * 128
    sc_hi = sc_lo + 16 * 128

    def transpose(g_ref, t_ref):
        for j in range(128):
            v0 = g_ref[j, pl.ds(0, 16)]
            v1 = g_ref[j, pl.ds(16, 16)]
            plsc.store_scatter(t_ref, [sc_lo + j], v0)
            plsc.store_scatter(t_ref, [sc_hi + j], v1)

    def run_phase(ids_hbm, out_hbm):
        for P in range(LT):
            pltpu.async_copy(ids_hbm.at[P, wid], idx_v.at[P], ssem)
        for P in range(LT):
            pltpu.make_async_copy(ids_hbm.at[0, wid], idx_v.at[P],
                                  ssem).wait()

        def fire_gather(l, g_ref):
            pltpu.async_copy(table_hbm.at[idx_v.at[l // 8, l % 8]], g_ref,
                             gsem)

        def wait_gather(g_ref):
            pltpu.make_async_copy(table_hbm.at[idx_v.at[0, 0]], g_ref,
                                  gsem).wait()

        def fire_wb(l, t_ref):
            for r in range(ET):
                pltpu.async_copy(t_ref.at[pl.ds(r * 1024, 1024)],
                                 out_hbm.at[l, r, wid], osem)

        def wait_wb(t_ref):
            for r in range(ET):
                pltpu.make_async_copy(t_ref.at[pl.ds(r * 1024, 1024)],
                                      out_hbm.at[0, r, wid], osem).wait()

        fire_gather(0, g0)

        def body(t, _):
            for p, g, tb, g_next in ((0, g0, t0, g1), (1, g1, t1, g0)):
                l = 2 * t + p
                wait_gather(g)

                def _fire_next(l=l, g_next=g_next):
                    fire_gather(l + 1, g_next)

                def _drain_tb(tb=tb):
                    wait_wb(tb)

                if p == 0:
                    _fire_next()
                else:
                    pl.when(l + 1 < L)(_fire_next)
                pl.when(t > 0)(_drain_tb)
                transpose(g, tb)
                fire_wb(l, tb)
            return 0

        lax.fori_loop(0, L // 2, body, 0, unroll=False)
        wait_wb(t0)
        wait_wb(t1)

    run_phase(ids0_hbm, out0_hbm)
    run_phase(ids1_hbm, out1_hbm)


def _to4(ids):
    return ids.astype(jnp.int32).T.reshape(LT, 8, NW, 128).transpose(0, 2, 1, 3)


def kernel(input_ids, mlm_input_ids, attention_mask, table):
    out0, out1 = _gather_kernel(_to4(input_ids), _to4(mlm_input_ids), table)

    def _from5(o4):
        o5 = o4.reshape(L, ET, NW, 8, 128)
        return o5.transpose(2, 4, 0, 1, 3).reshape(B, L, DIM)

    return (_from5(out0), _from5(out1), attention_mask)


# speedup vs baseline: 1.0673x; 1.0673x over previous
import functools

import jax
import jax.numpy as jnp
from jax import lax
from jax.experimental import pallas as pl
from jax.experimental.pallas import tpu as pltpu
from jax.experimental.pallas import tpu_sc as plsc

B = 4096
L = 200
DIM = 32

NC = 2
NS = 16
NW = NC * NS

LT = L // 8
ET = DIM // 8


@functools.partial(
    pl.kernel,
    mesh=plsc.VectorSubcoreMesh(core_axis_name="c", subcore_axis_name="s"),
    out_type=(
        jax.ShapeDtypeStruct((L, ET, NW, 1024), jnp.float32),
        jax.ShapeDtypeStruct((L, ET, NW, 1024), jnp.float32),
    ),
    scratch_types=[
        pltpu.VMEM((LT, 8, 128), jnp.int32),
        pltpu.VMEM((128, DIM), jnp.float32),
        pltpu.VMEM((128, DIM), jnp.float32),
        pltpu.VMEM((ET * 1024 + 8,), jnp.float32),
        pltpu.VMEM((ET * 1024 + 8,), jnp.float32),
        pltpu.SemaphoreType.DMA,
        pltpu.SemaphoreType.DMA,
        pltpu.SemaphoreType.DMA,
    ],
    compiler_params=pltpu.CompilerParams(use_tc_tiling_on_sc=False,
                                         needs_layout_passes=False),
)
def _gather_kernel(ids0_hbm, ids1_hbm, table_hbm, out0_hbm, out1_hbm,
                   idx_v, g0, g1, t0, t1, gsem, osem, ssem):
    wid = lax.axis_index("s") * NC + lax.axis_index("c")
    iota16 = lax.iota(jnp.int32, 16)
    sc_lo = iota16 * 128
    sc_d = [sc_lo + d for d in range(8)]
    SPAN = 15 * 128 + 9

    def transpose(g_ref, t_ref):
        for j0 in range(0, 128, 8):
            vs = []
            for dj in range(8):
                vs.append((dj, g_ref[j0 + dj, pl.ds(0, 16)],
                           g_ref[j0 + dj, pl.ds(16, 16)]))
            for dj, v0, v1 in vs:
                plsc.store_scatter(t_ref.at[pl.ds(j0, SPAN)], [sc_d[dj]],
                                   v0)
                plsc.store_scatter(t_ref.at[pl.ds(2048 + j0, SPAN)],
                                   [sc_d[dj]], v1)

    def run_phase(ids_hbm, out_hbm):
        for P in range(LT):
            pltpu.async_copy(ids_hbm.at[P, wid], idx_v.at[P], ssem)
        for P in range(LT):
            pltpu.make_async_copy(ids_hbm.at[0, wid], idx_v.at[P],
                                  ssem).wait()

        def fire_gather(l, g_ref):
            pltpu.async_copy(table_hbm.at[idx_v.at[l // 8, l % 8]], g_ref,
                             gsem)

        def wait_gather(g_ref):
            pltpu.make_async_copy(table_hbm.at[idx_v.at[0, 0]], g_ref,
                                  gsem).wait()

        def fire_wb(l, t_ref):
            for r in range(ET):
                pltpu.async_copy(t_ref.at[pl.ds(r * 1024, 1024)],
                                 out_hbm.at[l, r, wid], osem)

        def wait_wb(t_ref):
            for r in range(ET):
                pltpu.make_async_copy(t_ref.at[pl.ds(r * 1024, 1024)],
                                      out_hbm.at[0, r, wid], osem).wait()

        fire_gather(0, g0)

        def body(t, _):
            for p, g, tb, g_next in ((0, g0, t0, g1), (1, g1, t1, g0)):
                l = 2 * t + p
                wait_gather(g)

                def _fire_next(l=l, g_next=g_next):
                    fire_gather(l + 1, g_next)

                def _drain_tb(tb=tb):
                    wait_wb(tb)

                if p == 0:
                    _fire_next()
                else:
                    pl.when(l + 1 < L)(_fire_next)
                pl.when(t > 0)(_drain_tb)
                transpose(g, tb)
                fire_wb(l, tb)
            return 0

        lax.fori_loop(0, L // 2, body, 0, unroll=False)
        wait_wb(t0)
        wait_wb(t1)

    run_phase(ids0_hbm, out0_hbm)
    run_phase(ids1_hbm, out1_hbm)


def _to4(ids):
    return ids.astype(jnp.int32).T.reshape(LT, 8, NW, 128).transpose(0, 2, 1, 3)


def kernel(input_ids, mlm_input_ids, attention_mask, table):
    out0, out1 = _gather_kernel(_to4(input_ids), _to4(mlm_input_ids), table)

    def _from5(o4):
        o5 = o4.reshape(L, ET, NW, 8, 128)
        return o5.transpose(2, 4, 0, 1, 3).reshape(B, L, DIM)

    return (_from5(out0), _from5(out1), attention_mask)


# speedup vs baseline: 1.6977x; 1.5906x over previous
import functools

import jax
import jax.numpy as jnp
from jax import lax
from jax.experimental import pallas as pl
from jax.experimental.pallas import tpu as pltpu
from jax.experimental.pallas import tpu_sc as plsc

B = 4096
L = 200
DIM = 32

NC = 2
NS = 16
NW = NC * NS

LT = L // 8
ET = DIM // 8


@functools.partial(
    pl.kernel,
    mesh=plsc.VectorSubcoreMesh(core_axis_name="c", subcore_axis_name="s"),
    out_type=(
        jax.ShapeDtypeStruct((L, ET, NW, 1024), jnp.float32),
        jax.ShapeDtypeStruct((L, ET, NW, 1024), jnp.float32),
    ),
    scratch_types=[
        pltpu.VMEM((LT, 8, 128), jnp.int32),
        pltpu.VMEM((128, DIM), jnp.float32),
        pltpu.VMEM((128, DIM), jnp.float32),
        pltpu.VMEM((ET * 1024 + 8,), jnp.float32),
        pltpu.VMEM((ET * 1024 + 8,), jnp.float32),
        pltpu.SemaphoreType.DMA,
        pltpu.SemaphoreType.DMA,
        pltpu.SemaphoreType.DMA,
    ],
    compiler_params=pltpu.CompilerParams(use_tc_tiling_on_sc=False,
                                         needs_layout_passes=False),
)
def _gather_kernel(ids0_hbm, ids1_hbm, table_hbm, out0_hbm, out1_hbm,
                   idx_v, g0, g1, t0, t1, gsem, osem, ssem):
    wid = lax.axis_index("s") * NC + lax.axis_index("c")
    iota16 = lax.iota(jnp.int32, 16)
    perm = [(iota16 + s) & 15 for s in range(16)]
    dst_idx = [iota16 * 128 + perm[s] for s in range(16)]
    e_half = (iota16, iota16 + 16)

    def transpose(g_ref, t_ref):
        for eh, e_idx in enumerate(e_half):
            for j0 in range(0, 128, 16):
                src = g_ref.at[pl.ds(j0, 16)]
                dst = t_ref.at[pl.ds(eh * 2048 + j0, 15 * 128 + 16)]
                vs = [plsc.load_gather(src, [perm[s], e_idx])
                      for s in range(16)]
                for s in range(16):
                    plsc.store_scatter(dst, [dst_idx[s]], vs[s])

    def run_phase(ids_hbm, out_hbm):
        for P in range(LT):
            pltpu.async_copy(ids_hbm.at[P, wid], idx_v.at[P], ssem)
        for P in range(LT):
            pltpu.make_async_copy(ids_hbm.at[0, wid], idx_v.at[P],
                                  ssem).wait()

        def fire_gather(l, g_ref):
            pltpu.async_copy(table_hbm.at[idx_v.at[l // 8, l % 8]], g_ref,
                             gsem)

        def wait_gather(g_ref):
            pltpu.make_async_copy(table_hbm.at[idx_v.at[0, 0]], g_ref,
                                  gsem).wait()

        def fire_wb(l, t_ref):
            for r in range(ET):
                pltpu.async_copy(t_ref.at[pl.ds(r * 1024, 1024)],
                                 out_hbm.at[l, r, wid], osem)

        def wait_wb(t_ref):
            for r in range(ET):
                pltpu.make_async_copy(t_ref.at[pl.ds(r * 1024, 1024)],
                                      out_hbm.at[0, r, wid], osem).wait()

        fire_gather(0, g0)

        def body(t, _):
            for p, g, tb, g_next in ((0, g0, t0, g1), (1, g1, t1, g0)):
                l = 2 * t + p
                wait_gather(g)

                def _fire_next(l=l, g_next=g_next):
                    fire_gather(l + 1, g_next)

                def _drain_tb(tb=tb):
                    wait_wb(tb)

                if p == 0:
                    _fire_next()
                else:
                    pl.when(l + 1 < L)(_fire_next)
                pl.when(t > 0)(_drain_tb)
                transpose(g, tb)
                fire_wb(l, tb)
            return 0

        lax.fori_loop(0, L // 2, body, 0, unroll=False)
        wait_wb(t0)
        wait_wb(t1)

    run_phase(ids0_hbm, out0_hbm)
    run_phase(ids1_hbm, out1_hbm)


def _to4(ids):
    return ids.astype(jnp.int32).T.reshape(LT, 8, NW, 128).transpose(0, 2, 1, 3)


def kernel(input_ids, mlm_input_ids, attention_mask, table):
    out0, out1 = _gather_kernel(_to4(input_ids), _to4(mlm_input_ids), table)

    def _from5(o4):
        o5 = o4.reshape(L, ET, NW, 8, 128)
        return o5.transpose(2, 4, 0, 1, 3).reshape(B, L, DIM)

    return (_from5(out0), _from5(out1), attention_mask)


# speedup vs baseline: 1.9250x; 1.1339x over previous
import functools

import jax
import jax.numpy as jnp
from jax import lax
from jax.experimental import pallas as pl
from jax.experimental.pallas import tpu as pltpu
from jax.experimental.pallas import tpu_sc as plsc

B = 4096
L = 200
DIM = 32

NC = 2
NS = 16
NW = NC * NS

LT = L // 8
ET = DIM // 8


@functools.partial(
    pl.kernel,
    mesh=plsc.VectorSubcoreMesh(core_axis_name="c", subcore_axis_name="s"),
    out_type=(
        jax.ShapeDtypeStruct((L, ET, NW, 1024), jnp.float32),
        jax.ShapeDtypeStruct((L, ET, NW, 1024), jnp.float32),
    ),
    scratch_types=[
        pltpu.VMEM((LT, 8, 128), jnp.int32),
        pltpu.VMEM((128, DIM), jnp.float32),
        pltpu.VMEM((128, DIM), jnp.float32),
        pltpu.VMEM((ET * 1024 + 8,), jnp.float32),
        pltpu.VMEM((ET * 1024 + 8,), jnp.float32),
        pltpu.SemaphoreType.DMA,
        pltpu.SemaphoreType.DMA,
        pltpu.SemaphoreType.DMA,
    ],
    compiler_params=pltpu.CompilerParams(use_tc_tiling_on_sc=False,
                                         needs_layout_passes=False),
)
def _gather_kernel(ids0_hbm, ids1_hbm, table_hbm, out0_hbm, out1_hbm,
                   idx_v, g0, g1, t0, t1, gsem, osem, ssem):
    wid = lax.axis_index("s") * NC + lax.axis_index("c")
    iota16 = lax.iota(jnp.int32, 16)
    perm = [(iota16 + s) & 15 for s in range(16)]
    dst_idx = [iota16 * 128 + perm[s] for s in range(16)]
    e_half = (iota16, iota16 + 16)

    def transpose(g_ref, t_ref):
        for eh, e_idx in enumerate(e_half):
            for j0 in range(0, 128, 16):
                src = g_ref.at[pl.ds(j0, 16)]
                dst = t_ref.at[pl.ds(eh * 2048 + j0, 15 * 128 + 16)]
                vs = [plsc.load_gather(src, [perm[s], e_idx])
                      for s in range(16)]
                for s in range(16):
                    plsc.store_scatter(dst, [dst_idx[s]], vs[s])

    def run_phase(ids_hbm, out_hbm):
        for P in range(LT):
            pltpu.async_copy(ids_hbm.at[P, wid], idx_v.at[P], ssem)
        for P in range(LT):
            pltpu.make_async_copy(ids_hbm.at[0, wid], idx_v.at[P],
                                  ssem).wait()

        def fire_gather(l, g_ref):
            pltpu.async_copy(table_hbm.at[idx_v.at[l // 8, l % 8]], g_ref,
                             gsem)

        def wait_gather(g_ref):
            pltpu.make_async_copy(table_hbm.at[idx_v.at[0, 0]], g_ref,
                                  gsem).wait()

        def fire_wb(l, t_ref):
            for r in range(ET):
                pltpu.async_copy(t_ref.at[pl.ds(r * 1024, 1024)],
                                 out_hbm.at[l, r, wid], osem)

        def wait_wb(t_ref):
            for r in range(ET):
                pltpu.make_async_copy(t_ref.at[pl.ds(r * 1024, 1024)],
                                      out_hbm.at[0, r, wid], osem).wait()

        fire_gather(0, g0)

        def body(t, _):
            for p, g, tb, g_next in ((0, g0, t0, g1), (1, g1, t1, g0)):
                l = 2 * t + p

                def _fire_next(l=l, g_next=g_next):
                    fire_gather(l + 1, g_next)

                def _drain_tb(tb=tb):
                    wait_wb(tb)

                if p == 0:
                    _fire_next()
                else:
                    pl.when(l + 1 < L)(_fire_next)
                wait_gather(g)
                pl.when(t > 0)(_drain_tb)
                transpose(g, tb)
                fire_wb(l, tb)
            return 0

        lax.fori_loop(0, L // 2, body, 0, unroll=False)
        wait_wb(t0)
        wait_wb(t1)

    run_phase(ids0_hbm, out0_hbm)
    run_phase(ids1_hbm, out1_hbm)


def _to4(ids):
    return ids.astype(jnp.int32).T.reshape(LT, 8, NW, 128).transpose(0, 2, 1, 3)


def kernel(input_ids, mlm_input_ids, attention_mask, table):
    out0, out1 = _gather_kernel(_to4(input_ids), _to4(mlm_input_ids), table)

    def _from5(o4):
        o5 = o4.reshape(L, ET, NW, 8, 128)
        return o5.transpose(2, 4, 0, 1, 3).reshape(B, L, DIM)

    return (_from5(out0), _from5(out1), attention_mask)
